# speedup vs baseline: 1.2601x; 1.2601x over previous
import functools

import jax
import jax.numpy as jnp
from jax import lax
from jax.experimental import pallas as pl
from jax.experimental.pallas import tpu as pltpu
from jax.experimental.pallas import tpu_sc as plsc

_N = 10000
_K = 32
_C = 128
_NW = 32
_NPW = 320
_NPAD = _NW * _NPW
_CH = 4
_NCH = _NPW // _CH
_NV = _C // 16


def _sc_max_rel(x_rows, src_flat, dst_flat):
    mesh = plsc.VectorSubcoreMesh(core_axis_name="c", subcore_axis_name="s")

    @functools.partial(
        pl.kernel,
        mesh=mesh,
        out_type=jax.ShapeDtypeStruct((_NPAD, _C), jnp.float32),
        scratch_types=[
            pltpu.VMEM((_CH * _K,), jnp.int32),
            pltpu.VMEM((_CH * _K,), jnp.int32),
            pltpu.VMEM((_CH * _K, _C), jnp.float32),
            pltpu.VMEM((_CH * _K, _C), jnp.float32),
            pltpu.VMEM((_CH, _C), jnp.float32),
            pltpu.SemaphoreType.DMA,
            pltpu.SemaphoreType.DMA,
        ],
    )
    def sc_kernel(x_hbm, src_hbm, dst_hbm, out_hbm,
                  idx_s, idx_d, rows_s, rows_d, outc, sem_s, sem_d):
        wid = lax.axis_index("s") * 2 + lax.axis_index("c")
        base = wid * _NPW

        def chunk_body(ci, carry):
            node0 = base + ci * _CH
            off = node0 * _K
            pltpu.sync_copy(src_hbm.at[pl.ds(off, _CH * _K)], idx_s)
            pltpu.sync_copy(dst_hbm.at[pl.ds(off, _CH * _K)], idx_d)
            cs = pltpu.async_copy(x_hbm.at[idx_s], rows_s, sem_s)
            cd = pltpu.async_copy(x_hbm.at[idx_d], rows_d, sem_d)
            cs.wait()
            cd.wait()
            for i in range(_CH):
                r0 = i * _K

                def k_body(k, accs, r0=r0):
                    r = r0 + k
                    out = []
                    for v in range(_NV):
                        s = rows_s[r, pl.ds(v * 16, 16)]
                        d = rows_d[r, pl.ds(v * 16, 16)]
                        out.append(jnp.maximum(accs[v], s - d))
                    return tuple(out)

                init = tuple(
                    rows_s[r0, pl.ds(v * 16, 16)] - rows_d[r0, pl.ds(v * 16, 16)]
                    for v in range(_NV))
                accs = lax.fori_loop(1, _K, k_body, init)
                for v in range(_NV):
                    outc[i, pl.ds(v * 16, 16)] = accs[v]
            pltpu.sync_copy(outc, out_hbm.at[pl.ds(node0, _CH)])
            return carry

        lax.fori_loop(0, _NCH, chunk_body, 0)

    return sc_kernel(x_rows, src_flat, dst_flat)


def _tc_fuse(x2, xmax_rows, a, b, gamma, beta):
    def body(x2_ref, xm_ref, a_ref, b_ref, g_ref, bt_ref, o_ref):
        xp = x2_ref[...]
        xm = xm_ref[pl.ds(0, _N), :]
        y = lax.dot_general(a_ref[...], xp, (((1,), (0,)), ((), ())),
                            precision=lax.Precision.HIGHEST,
                            preferred_element_type=jnp.float32)
        y = y + lax.dot_general(b_ref[...], xm, (((1,), (1,)), ((), ())),
                                precision=lax.Precision.HIGHEST,
                                preferred_element_type=jnp.float32)
        m = jnp.mean(y, axis=1, keepdims=True)
        yc = y - m
        var = jnp.mean(yc * yc, axis=1, keepdims=True)
        scale = g_ref[...] * lax.rsqrt(var + 1e-5)
        o_ref[...] = jnp.maximum(yc * scale + bt_ref[...], 0.0)

    return pl.pallas_call(
        body,
        out_shape=jax.ShapeDtypeStruct((_C, _N), jnp.float32),
    )(x2, xmax_rows, a, b, gamma, beta)


def kernel(x, edge_index, conv_w, conv_b, bn_gamma, bn_beta):
    del conv_b
    x2 = x[0, :, :, 0]
    x_rows = x2.T
    ei = jnp.pad(edge_index, ((0, 0), (0, _NPAD - _N), (0, 0)))
    src_flat = ei[0].reshape(-1)
    dst_flat = ei[1].reshape(-1)
    xmax_rows = _sc_max_rel(x_rows, src_flat, dst_flat)
    w = conv_w.reshape(4, 32, 32, 2)
    eye = jnp.eye(4, dtype=conv_w.dtype)
    a = jnp.einsum('goc,gh->gohc', w[..., 0], eye).reshape(_C, _C)
    b = jnp.einsum('goc,gh->gohc', w[..., 1], eye).reshape(_C, _C)
    y = _tc_fuse(x2, xmax_rows, a, b,
                 bn_gamma.reshape(_C, 1), bn_beta.reshape(_C, 1))
    return y[None, :, :, None]


# speedup vs baseline: 1.4065x; 1.1162x over previous
import functools

import jax
import jax.numpy as jnp
from jax import lax
from jax.experimental import pallas as pl
from jax.experimental.pallas import tpu as pltpu
from jax.experimental.pallas import tpu_sc as plsc

_N = 10000
_K = 32
_C = 128
_NW = 32
_NPW = 320
_NPAD = _NW * _NPW
_CH = 4
_NCH = _NPW // _CH
_NV = _C // 16


_CHK = _CH * _K


def _sc_max_rel(x_rows, src_flat, dst_flat):
    mesh = plsc.VectorSubcoreMesh(core_axis_name="c", subcore_axis_name="s")

    @functools.partial(
        pl.kernel,
        mesh=mesh,
        out_type=jax.ShapeDtypeStruct((_NPAD, _C), jnp.float32),
        scratch_types=[
            pltpu.VMEM((_NPW * _K,), jnp.int32),
            pltpu.VMEM((_NPW * _K,), jnp.int32),
            pltpu.VMEM((_CHK, _C), jnp.float32),
            pltpu.VMEM((_CHK, _C), jnp.float32),
            pltpu.VMEM((_CHK, _C), jnp.float32),
            pltpu.VMEM((_CHK, _C), jnp.float32),
            pltpu.VMEM((_NPW, _C), jnp.float32),
            pltpu.SemaphoreType.DMA,
            pltpu.SemaphoreType.DMA,
            pltpu.SemaphoreType.DMA,
            pltpu.SemaphoreType.DMA,
        ],
    )
    def sc_kernel(x_hbm, src_hbm, dst_hbm, out_hbm,
                  idx_s, idx_d, rs0, rs1, rd0, rd1, out_all,
                  sem_s0, sem_s1, sem_d0, sem_d1):
        wid = lax.axis_index("s") * 2 + lax.axis_index("c")
        base = wid * _NPW
        rs = (rs0, rs1)
        rd = (rd0, rd1)
        sem_s = (sem_s0, sem_s1)
        sem_d = (sem_d0, sem_d1)

        pltpu.sync_copy(src_hbm.at[pl.ds(base * _K, _NPW * _K)], idx_s)
        pltpu.sync_copy(dst_hbm.at[pl.ds(base * _K, _NPW * _K)], idx_d)

        def fire(ci, b):
            off = ci * _CHK
            pltpu.async_copy(x_hbm.at[idx_s.at[pl.ds(off, _CHK)]],
                             rs[b], sem_s[b])
            pltpu.async_copy(x_hbm.at[idx_d.at[pl.ds(off, _CHK)]],
                             rd[b], sem_d[b])

        def wait_rows(b):
            dummy = x_hbm.at[pl.ds(0, _CHK)]
            pltpu.make_async_copy(dummy, rs[b], sem_s[b]).wait()
            pltpu.make_async_copy(dummy, rd[b], sem_d[b]).wait()

        def compute(ci, b):
            for i in range(_CH):
                r0 = i * _K

                def k_body(k, accs, r0=r0, b=b):
                    r = r0 + k
                    out = []
                    for v in range(_NV):
                        s = rs[b][r, pl.ds(v * 16, 16)]
                        d = rd[b][r, pl.ds(v * 16, 16)]
                        out.append(jnp.maximum(accs[v], s - d))
                    return tuple(out)

                init = tuple(
                    rs[b][r0, pl.ds(v * 16, 16)] - rd[b][r0, pl.ds(v * 16, 16)]
                    for v in range(_NV))
                accs = lax.fori_loop(1, _K, k_body, init)
                node = ci * _CH + i
                for v in range(_NV):
                    out_all[node, pl.ds(v * 16, 16)] = accs[v]

        fire(0, 0)

        def pair_body(p, carry):
            for j in range(2):
                ci = 2 * p + j
                wait_rows(j)

                @pl.when(ci + 1 < _NCH)
                def _(ci=ci, j=j):
                    fire(ci + 1, 1 - j)

                compute(ci, j)
            return carry

        lax.fori_loop(0, _NCH // 2, pair_body, 0)
        pltpu.sync_copy(out_all, out_hbm.at[pl.ds(base, _NPW)])

    return sc_kernel(x_rows, src_flat, dst_flat)


def _tc_fuse(x2, xmax_rows, a, b, gamma, beta):
    def body(x2_ref, xm_ref, a_ref, b_ref, g_ref, bt_ref, o_ref):
        xp = x2_ref[...]
        xm = xm_ref[pl.ds(0, _N), :]
        y = lax.dot_general(a_ref[...], xp, (((1,), (0,)), ((), ())),
                            precision=lax.Precision.HIGHEST,
                            preferred_element_type=jnp.float32)
        y = y + lax.dot_general(b_ref[...], xm, (((1,), (1,)), ((), ())),
                                precision=lax.Precision.HIGHEST,
                                preferred_element_type=jnp.float32)
        m = jnp.mean(y, axis=1, keepdims=True)
        yc = y - m
        var = jnp.mean(yc * yc, axis=1, keepdims=True)
        scale = g_ref[...] * lax.rsqrt(var + 1e-5)
        o_ref[...] = jnp.maximum(yc * scale + bt_ref[...], 0.0)

    return pl.pallas_call(
        body,
        out_shape=jax.ShapeDtypeStruct((_C, _N), jnp.float32),
    )(x2, xmax_rows, a, b, gamma, beta)


def kernel(x, edge_index, conv_w, conv_b, bn_gamma, bn_beta):
    del conv_b
    x2 = x[0, :, :, 0]
    x_rows = x2.T
    ei = jnp.pad(edge_index, ((0, 0), (0, _NPAD - _N), (0, 0)))
    src_flat = ei[0].reshape(-1)
    dst_flat = ei[1].reshape(-1)
    xmax_rows = _sc_max_rel(x_rows, src_flat, dst_flat)
    w = conv_w.reshape(4, 32, 32, 2)
    eye = jnp.eye(4, dtype=conv_w.dtype)
    a = jnp.einsum('goc,gh->gohc', w[..., 0], eye).reshape(_C, _C)
    b = jnp.einsum('goc,gh->gohc', w[..., 1], eye).reshape(_C, _C)
    y = _tc_fuse(x2, xmax_rows, a, b,
                 bn_gamma.reshape(_C, 1), bn_beta.reshape(_C, 1))
    return y[None, :, :, None]


# speedup vs baseline: 3.0446x; 2.1646x over previous
import functools

import jax
import jax.numpy as jnp
from jax import lax
from jax.experimental import pallas as pl
from jax.experimental.pallas import tpu as pltpu
from jax.experimental.pallas import tpu_sc as plsc

_N = 10000
_K = 32
_C = 128
_NSC = 2
_NT = 16
_CPT = _C // _NT
_NPAD = 10240
_NPC = _NPAD // _NSC
_CN = 128
_NCHK = _NPC // _CN
_G = _CN // 16


def _sc_max_rel(x_shard, idx_prep):
    mesh = plsc.VectorSubcoreMesh(core_axis_name="c", subcore_axis_name="s")

    @functools.partial(
        pl.kernel,
        mesh=mesh,
        out_type=jax.ShapeDtypeStruct((_NSC, _NT, _NCHK, _CPT, _CN),
                                      jnp.float32),
        compiler_params=pltpu.CompilerParams(needs_layout_passes=False),
        scratch_types=[
            pltpu.VMEM((_NPAD * _CPT,), jnp.float32),
            pltpu.VMEM((2, _K, _CN), jnp.int32),
            pltpu.VMEM((2, _K, _CN), jnp.int32),
            pltpu.VMEM((_CPT, _CN), jnp.float32),
            pltpu.VMEM((_CPT, _CN), jnp.float32),
            pltpu.SemaphoreType.DMA,
            pltpu.SemaphoreType.DMA,
            pltpu.SemaphoreType.DMA,
            pltpu.SemaphoreType.DMA,
        ],
    )
    def sc_kernel(x_hbm, idx_hbm, out_hbm, table, ib0, ib1, ob0, ob1,
                  si0, si1, so0, so1):
        c = lax.axis_index("c")
        t = lax.axis_index("s")
        ib = (ib0, ib1)
        ob = (ob0, ob1)
        si = (si0, si1)
        so = (so0, so1)
        ch_vecs = [jnp.full((16,), ch, jnp.int32) for ch in range(_CPT)]
        eight = jnp.full((16,), _CPT, jnp.int32)

        pltpu.sync_copy(x_hbm.at[t], table)

        def fire_idx(ci, j):
            pltpu.async_copy(idx_hbm.at[c, ci], ib[j], si[j])

        def wait_idx(j):
            pltpu.make_async_copy(idx_hbm.at[c, 0], ib[j], si[j]).wait()

        def fire_out(ci, j):
            pltpu.async_copy(ob[j], out_hbm.at[c, t, ci], so[j])

        def wait_out(j):
            pltpu.make_async_copy(ob[j], out_hbm.at[c, t, 0], so[j]).wait()

        def compute(j):
            ib_ = ib[j]
            ob_ = ob[j]

            def group_body(g, carry):
                n0 = g * 16
                accs = None
                for k in range(_K):
                    isv = ib_[0, k, pl.ds(n0, 16)] * eight
                    idv = ib_[1, k, pl.ds(n0, 16)] * eight
                    new = []
                    for ch in range(_CPT):
                        s = plsc.load_gather(table, [isv + ch_vecs[ch]])
                        d = plsc.load_gather(table, [idv + ch_vecs[ch]])
                        diff = s - d
                        if accs is None:
                            new.append(diff)
                        else:
                            new.append(jnp.maximum(accs[ch], diff))
                    accs = new
                for ch in range(_CPT):
                    ob_[ch, pl.ds(n0, 16)] = accs[ch]
                return carry

            lax.fori_loop(0, _G, group_body, 0)

        fire_idx(0, 0)

        def pair_body(p, carry):
            for j in range(2):
                ci = 2 * p + j
                wait_idx(j)

                @pl.when(ci + 1 < _NCHK)
                def _(ci=ci, j=j):
                    fire_idx(ci + 1, 1 - j)

                @pl.when(p >= 1)
                def _(j=j):
                    wait_out(j)

                compute(j)
                fire_out(ci, j)
            return carry

        lax.fori_loop(0, _NCHK // 2, pair_body, 0)
        wait_out(0)
        wait_out(1)

    return sc_kernel(x_shard, idx_prep)


def _tc_fuse(x2, xmax_rows, a, b, gamma, beta):
    def body(x2_ref, xm_ref, a_ref, b_ref, g_ref, bt_ref, o_ref):
        xp = x2_ref[...]
        xm = xm_ref[pl.ds(0, _N), :]
        y = lax.dot_general(a_ref[...], xp, (((1,), (0,)), ((), ())),
                            precision=lax.Precision.HIGHEST,
                            preferred_element_type=jnp.float32)
        y = y + lax.dot_general(b_ref[...], xm, (((1,), (1,)), ((), ())),
                                precision=lax.Precision.HIGHEST,
                                preferred_element_type=jnp.float32)
        m = jnp.mean(y, axis=1, keepdims=True)
        yc = y - m
        var = jnp.mean(yc * yc, axis=1, keepdims=True)
        scale = g_ref[...] * lax.rsqrt(var + 1e-5)
        o_ref[...] = jnp.maximum(yc * scale + bt_ref[...], 0.0)

    return pl.pallas_call(
        body,
        out_shape=jax.ShapeDtypeStruct((_C, _N), jnp.float32),
    )(x2, xmax_rows, a, b, gamma, beta)


def kernel(x, edge_index, conv_w, conv_b, bn_gamma, bn_beta):
    del conv_b
    x2 = x[0, :, :, 0]
    x_pad = jnp.pad(x2.T, ((0, _NPAD - _N), (0, 0)))
    x_shard = (x_pad.reshape(_NPAD, _NT, _CPT).transpose(1, 0, 2)
               .reshape(_NT, _NPAD * _CPT))
    ei_p = jnp.pad(edge_index, ((0, 0), (0, _NPAD - _N), (0, 0)))
    idx_prep = (ei_p.transpose(0, 2, 1)
                .reshape(2, _K, _NSC, _NCHK, _CN)
                .transpose(2, 3, 0, 1, 4))
    outp = _sc_max_rel(x_shard, idx_prep)
    xmax_rows = outp.transpose(0, 2, 4, 1, 3).reshape(_NPAD, _C)
    w = conv_w.reshape(4, 32, 32, 2)
    eye = jnp.eye(4, dtype=conv_w.dtype)
    a = jnp.einsum('goc,gh->gohc', w[..., 0], eye).reshape(_C, _C)
    b = jnp.einsum('goc,gh->gohc', w[..., 1], eye).reshape(_C, _C)
    y = _tc_fuse(x2, xmax_rows, a, b,
                 bn_gamma.reshape(_C, 1), bn_beta.reshape(_C, 1))
    return y[None, :, :, None]


# speedup vs baseline: 6.0709x; 1.9940x over previous
import functools

import jax
import jax.numpy as jnp
from jax import lax
from jax.experimental import pallas as pl
from jax.experimental.pallas import tpu as pltpu
from jax.experimental.pallas import tpu_sc as plsc

_N = 10000
_K = 32
_C = 128
_NSC = 2
_NT = 16
_CPT = _C // _NT
_WPT = _CPT // 2
_NPAD = 10240
_NPC = _NPAD // _NSC
_CN = 128
_NCHK = _NPC // _CN
_G = _CN // 16


def _sc_max_rel(x_shard, idx_prep):
    mesh = plsc.VectorSubcoreMesh(core_axis_name="c", subcore_axis_name="s")

    @functools.partial(
        pl.kernel,
        mesh=mesh,
        out_type=jax.ShapeDtypeStruct((_NSC, _NT, _NCHK, _WPT, _CN),
                                      jnp.int32),
        compiler_params=pltpu.CompilerParams(needs_layout_passes=False),
        scratch_types=[
            pltpu.VMEM((_NPAD * _WPT,), jnp.int32),
            pltpu.VMEM((2, _K, _CN), jnp.int32),
            pltpu.VMEM((2, _K, _CN), jnp.int32),
            pltpu.VMEM((_WPT, _CN), jnp.int32),
            pltpu.VMEM((_WPT, _CN), jnp.int32),
            pltpu.SemaphoreType.DMA,
            pltpu.SemaphoreType.DMA,
            pltpu.SemaphoreType.DMA,
            pltpu.SemaphoreType.DMA,
        ],
    )
    def sc_kernel(x_hbm, idx_hbm, out_hbm, table, ib0, ib1, ob0, ob1,
                  si0, si1, so0, so1):
        c = lax.axis_index("c")
        t = lax.axis_index("s")
        ib = (ib0, ib1)
        ob = (ob0, ob1)
        si = (si0, si1)
        so = (so0, so1)
        w_vecs = [jnp.full((16,), w, jnp.int32) for w in range(_WPT)]
        four = jnp.full((16,), _WPT, jnp.int32)

        pltpu.sync_copy(x_hbm.at[t], table)

        def fire_idx(ci, j):
            pltpu.async_copy(idx_hbm.at[c, ci], ib[j], si[j])

        def wait_idx(j):
            pltpu.make_async_copy(idx_hbm.at[c, 0], ib[j], si[j]).wait()

        def fire_out(ci, j):
            pltpu.async_copy(ob[j], out_hbm.at[c, t, ci], so[j])

        def wait_out(j):
            pltpu.make_async_copy(ob[j], out_hbm.at[c, t, 0], so[j]).wait()

        def compute(j):
            ib_ = ib[j]
            ob_ = ob[j]

            def group_body(g, carry):
                n0 = g * 16
                accs = None
                for k in range(_K):
                    isv = ib_[0, k, pl.ds(n0, 16)] * four
                    idv = ib_[1, k, pl.ds(n0, 16)] * four
                    new = []
                    for w in range(_WPT):
                        s = plsc.bitcast(
                            plsc.load_gather(table, [isv + w_vecs[w]]),
                            jnp.bfloat16)
                        d = plsc.bitcast(
                            plsc.load_gather(table, [idv + w_vecs[w]]),
                            jnp.bfloat16)
                        diff = s - d
                        if accs is None:
                            new.append(diff)
                        else:
                            new.append(jnp.maximum(accs[w], diff))
                    accs = new
                for w in range(_WPT):
                    ob_[w, pl.ds(n0, 16)] = plsc.bitcast(accs[w], jnp.int32)
                return carry

            lax.fori_loop(0, _G, group_body, 0)

        fire_idx(0, 0)

        def pair_body(p, carry):
            for j in range(2):
                ci = 2 * p + j
                wait_idx(j)

                @pl.when(ci + 1 < _NCHK)
                def _(ci=ci, j=j):
                    fire_idx(ci + 1, 1 - j)

                @pl.when(p >= 1)
                def _(j=j):
                    wait_out(j)

                compute(j)
                fire_out(ci, j)
            return carry

        lax.fori_loop(0, _NCHK // 2, pair_body, 0)
        wait_out(0)
        wait_out(1)

    return sc_kernel(x_shard, idx_prep)


def _tc_fuse(x2, xmax_rows, a, b, gamma, beta):
    def body(x2_ref, xm_ref, a_ref, b_ref, g_ref, bt_ref, o_ref):
        xp = x2_ref[...]
        xm = xm_ref[pl.ds(0, _N), :].astype(jnp.float32)
        y = lax.dot_general(a_ref[...], xp, (((1,), (0,)), ((), ())),
                            precision=lax.Precision.HIGHEST,
                            preferred_element_type=jnp.float32)
        y = y + lax.dot_general(b_ref[...], xm, (((1,), (1,)), ((), ())),
                                precision=lax.Precision.HIGHEST,
                                preferred_element_type=jnp.float32)
        m = jnp.mean(y, axis=1, keepdims=True)
        yc = y - m
        var = jnp.mean(yc * yc, axis=1, keepdims=True)
        scale = g_ref[...] * lax.rsqrt(var + 1e-5)
        o_ref[...] = jnp.maximum(yc * scale + bt_ref[...], 0.0)

    return pl.pallas_call(
        body,
        out_shape=jax.ShapeDtypeStruct((_C, _N), jnp.float32),
    )(x2, xmax_rows, a, b, gamma, beta)


def kernel(x, edge_index, conv_w, conv_b, bn_gamma, bn_beta):
    del conv_b
    x2 = x[0, :, :, 0]
    x_pad = jnp.pad(x2.T, ((0, _NPAD - _N), (0, 0)))
    x_pack = lax.bitcast_convert_type(
        x_pad.astype(jnp.bfloat16).reshape(_NPAD, _C // 2, 2), jnp.int32)
    x_shard = (x_pack.reshape(_NPAD, _NT, _WPT).transpose(1, 0, 2)
               .reshape(_NT, _NPAD * _WPT))
    ei_p = jnp.pad(edge_index, ((0, 0), (0, _NPAD - _N), (0, 0)))
    idx_prep = (ei_p.transpose(0, 2, 1)
                .reshape(2, _K, _NSC, _NCHK, _CN)
                .transpose(2, 3, 0, 1, 4))
    outp = _sc_max_rel(x_shard, idx_prep)
    xmax_rows = lax.bitcast_convert_type(
        outp.transpose(0, 2, 4, 1, 3), jnp.bfloat16).reshape(_NPAD, _C)
    w = conv_w.reshape(4, 32, 32, 2)
    eye = jnp.eye(4, dtype=conv_w.dtype)
    a = jnp.einsum('goc,gh->gohc', w[..., 0], eye).reshape(_C, _C)
    b = jnp.einsum('goc,gh->gohc', w[..., 1], eye).reshape(_C, _C)
    y = _tc_fuse(x2, xmax_rows, a, b,
                 bn_gamma.reshape(_C, 1), bn_beta.reshape(_C, 1))
    return y[None, :, :, None]


# speedup vs baseline: 7.3645x; 1.2131x over previous
import functools

import jax
import jax.numpy as jnp
from jax import lax
from jax.experimental import pallas as pl
from jax.experimental.pallas import tpu as pltpu
from jax.experimental.pallas import tpu_sc as plsc

_N = 10000
_K = 32
_C = 128
_NSC = 2
_NT = 16
_CPT = _C // _NT
_WPT = _CPT // 2
_NPAD = 10240
_NPC = _NPAD // _NSC
_CN = 128
_NCHK = _NPC // _CN
_G = _CN // 16


def _sc_max_rel(x_shard, idx_prep):
    mesh = plsc.VectorSubcoreMesh(core_axis_name="c", subcore_axis_name="s")

    @functools.partial(
        pl.kernel,
        mesh=mesh,
        out_type=jax.ShapeDtypeStruct((_NSC, _NT, _NCHK, _WPT, _CN),
                                      jnp.int32),
        compiler_params=pltpu.CompilerParams(needs_layout_passes=False),
        scratch_types=[
            pltpu.VMEM((_NPAD * _WPT,), jnp.int32),
            pltpu.VMEM((2, _K, _CN), jnp.int32),
            pltpu.VMEM((2, _K, _CN), jnp.int32),
            pltpu.VMEM((_WPT, _CN), jnp.int32),
            pltpu.VMEM((_WPT, _CN), jnp.int32),
            pltpu.SemaphoreType.DMA,
            pltpu.SemaphoreType.DMA,
            pltpu.SemaphoreType.DMA,
            pltpu.SemaphoreType.DMA,
        ],
    )
    def sc_kernel(x_hbm, idx_hbm, out_hbm, table, ib0, ib1, ob0, ob1,
                  si0, si1, so0, so1):
        c = lax.axis_index("c")
        t = lax.axis_index("s")
        ib = (ib0, ib1)
        ob = (ob0, ob1)
        si = (si0, si1)
        so = (so0, so1)
        w_vecs = [jnp.full((16,), w * _NPAD, jnp.int32) for w in range(_WPT)]

        pltpu.sync_copy(x_hbm.at[t], table)

        def fire_idx(ci, j):
            pltpu.async_copy(idx_hbm.at[c, ci], ib[j], si[j])

        def wait_idx(j):
            pltpu.make_async_copy(idx_hbm.at[c, 0], ib[j], si[j]).wait()

        def fire_out(ci, j):
            pltpu.async_copy(ob[j], out_hbm.at[c, t, ci], so[j])

        def wait_out(j):
            pltpu.make_async_copy(ob[j], out_hbm.at[c, t, 0], so[j]).wait()

        def compute(j):
            ib_ = ib[j]
            ob_ = ob[j]

            def group_body(g, carry):
                n0 = g * 16
                accs = None
                for k in range(_K):
                    isv = ib_[0, k, pl.ds(n0, 16)]
                    idv = ib_[1, k, pl.ds(n0, 16)]
                    new = []
                    for w in range(_WPT):
                        s = plsc.bitcast(
                            plsc.load_gather(table, [isv + w_vecs[w]]),
                            jnp.bfloat16)
                        d = plsc.bitcast(
                            plsc.load_gather(table, [idv + w_vecs[w]]),
                            jnp.bfloat16)
                        diff = s - d
                        if accs is None:
                            new.append(diff)
                        else:
                            new.append(jnp.maximum(accs[w], diff))
                    accs = new
                for w in range(_WPT):
                    ob_[w, pl.ds(n0, 16)] = plsc.bitcast(accs[w], jnp.int32)
                return carry

            lax.fori_loop(0, _G, group_body, 0)

        fire_idx(0, 0)

        def pair_body(p, carry):
            for j in range(2):
                ci = 2 * p + j
                wait_idx(j)

                @pl.when(ci + 1 < _NCHK)
                def _(ci=ci, j=j):
                    fire_idx(ci + 1, 1 - j)

                @pl.when(p >= 1)
                def _(j=j):
                    wait_out(j)

                compute(j)
                fire_out(ci, j)
            return carry

        lax.fori_loop(0, _NCHK // 2, pair_body, 0)
        wait_out(0)
        wait_out(1)

    return sc_kernel(x_shard, idx_prep)


def _tc_fuse(x2, xmax_rows, a, b, gamma, beta):
    def body(x2_ref, xm_ref, a_ref, b_ref, g_ref, bt_ref, o_ref):
        xp = x2_ref[...]
        xm = xm_ref[pl.ds(0, _N), :].astype(jnp.float32)
        y = lax.dot_general(a_ref[...], xp, (((1,), (0,)), ((), ())),
                            precision=lax.Precision.HIGHEST,
                            preferred_element_type=jnp.float32)
        y = y + lax.dot_general(b_ref[...], xm, (((1,), (1,)), ((), ())),
                                precision=lax.Precision.HIGHEST,
                                preferred_element_type=jnp.float32)
        m = jnp.mean(y, axis=1, keepdims=True)
        yc = y - m
        var = jnp.mean(yc * yc, axis=1, keepdims=True)
        scale = g_ref[...] * lax.rsqrt(var + 1e-5)
        o_ref[...] = jnp.maximum(yc * scale + bt_ref[...], 0.0)

    return pl.pallas_call(
        body,
        out_shape=jax.ShapeDtypeStruct((_C, _N), jnp.float32),
    )(x2, xmax_rows, a, b, gamma, beta)


def kernel(x, edge_index, conv_w, conv_b, bn_gamma, bn_beta):
    del conv_b
    x2 = x[0, :, :, 0]
    x_pad = jnp.pad(x2.T, ((0, _NPAD - _N), (0, 0)))
    x_pack = lax.bitcast_convert_type(
        x_pad.astype(jnp.bfloat16).reshape(_NPAD, _C // 2, 2), jnp.int32)
    x_shard = (x_pack.reshape(_NPAD, _NT, _WPT).transpose(1, 2, 0)
               .reshape(_NT, _NPAD * _WPT))
    ei_p = jnp.pad(edge_index, ((0, 0), (0, _NPAD - _N), (0, 0)))
    idx_prep = (ei_p.transpose(0, 2, 1)
                .reshape(2, _K, _NSC, _NCHK, _CN)
                .transpose(2, 3, 0, 1, 4))
    outp = _sc_max_rel(x_shard, idx_prep)
    xmax_rows = lax.bitcast_convert_type(
        outp.transpose(0, 2, 4, 1, 3), jnp.bfloat16).reshape(_NPAD, _C)
    w = conv_w.reshape(4, 32, 32, 2)
    eye = jnp.eye(4, dtype=conv_w.dtype)
    a = jnp.einsum('goc,gh->gohc', w[..., 0], eye).reshape(_C, _C)
    b = jnp.einsum('goc,gh->gohc', w[..., 1], eye).reshape(_C, _C)
    y = _tc_fuse(x2, xmax_rows, a, b,
                 bn_gamma.reshape(_C, 1), bn_beta.reshape(_C, 1))
    return y[None, :, :, None]


# speedup vs baseline: 7.5429x; 1.0242x over previous
import functools

import jax
import jax.numpy as jnp
from jax import lax
from jax.experimental import pallas as pl
from jax.experimental.pallas import tpu as pltpu
from jax.experimental.pallas import tpu_sc as plsc

_N = 10000
_K = 32
_C = 128
_NSC = 2
_NT = 16
_CPT = _C // _NT
_WPT = _CPT // 2
_NPAD = 10240
_NPC = _NPAD // _NSC
_CN = 128
_NCHK = _NPC // _CN
_G = _CN // 16


def _sc_max_rel(x_shard, idx_prep):
    mesh = plsc.VectorSubcoreMesh(core_axis_name="c", subcore_axis_name="s")

    @functools.partial(
        pl.kernel,
        mesh=mesh,
        out_type=jax.ShapeDtypeStruct((_NT, _WPT, _NSC, _NCHK, _CN),
                                      jnp.int32),
        compiler_params=pltpu.CompilerParams(needs_layout_passes=False),
        scratch_types=[
            pltpu.VMEM((_NPAD * _WPT,), jnp.int32),
            pltpu.VMEM((2, _K, _CN), jnp.int32),
            pltpu.VMEM((2, _K, _CN), jnp.int32),
            pltpu.VMEM((_WPT, _CN), jnp.int32),
            pltpu.VMEM((_WPT, _CN), jnp.int32),
            pltpu.SemaphoreType.DMA,
            pltpu.SemaphoreType.DMA,
            pltpu.SemaphoreType.DMA,
            pltpu.SemaphoreType.DMA,
        ],
    )
    def sc_kernel(x_hbm, idx_hbm, out_hbm, table, ib0, ib1, ob0, ob1,
                  si0, si1, so0, so1):
        c = lax.axis_index("c")
        t = lax.axis_index("s")
        ib = (ib0, ib1)
        ob = (ob0, ob1)
        si = (si0, si1)
        so = (so0, so1)
        w_vecs = [jnp.full((16,), w * _NPAD, jnp.int32) for w in range(_WPT)]

        pltpu.sync_copy(x_hbm.at[t], table)

        def fire_idx(ci, j):
            pltpu.async_copy(idx_hbm.at[c, ci], ib[j], si[j])

        def wait_idx(j):
            pltpu.make_async_copy(idx_hbm.at[c, 0], ib[j], si[j]).wait()

        def fire_out(ci, j):
            pltpu.async_copy(ob[j], out_hbm.at[t, :, c, ci], so[j])

        def wait_out(j):
            pltpu.make_async_copy(ob[j], out_hbm.at[t, :, c, 0], so[j]).wait()

        def compute(j):
            ib_ = ib[j]
            ob_ = ob[j]

            def group_body(g, carry):
                n0 = g * 16
                accs = None
                for k in range(_K):
                    isv = ib_[0, k, pl.ds(n0, 16)]
                    idv = ib_[1, k, pl.ds(n0, 16)]
                    new = []
                    for w in range(_WPT):
                        s = plsc.bitcast(
                            plsc.load_gather(table, [isv + w_vecs[w]]),
                            jnp.bfloat16)
                        d = plsc.bitcast(
                            plsc.load_gather(table, [idv + w_vecs[w]]),
                            jnp.bfloat16)
                        diff = s - d
                        if accs is None:
                            new.append(diff)
                        else:
                            new.append(jnp.maximum(accs[w], diff))
                    accs = new
                for w in range(_WPT):
                    ob_[w, pl.ds(n0, 16)] = plsc.bitcast(accs[w], jnp.int32)
                return carry

            lax.fori_loop(0, _G, group_body, 0)

        fire_idx(0, 0)

        def pair_body(p, carry):
            for j in range(2):
                ci = 2 * p + j
                wait_idx(j)

                @pl.when(ci + 1 < _NCHK)
                def _(ci=ci, j=j):
                    fire_idx(ci + 1, 1 - j)

                @pl.when(p >= 1)
                def _(j=j):
                    wait_out(j)

                compute(j)
                fire_out(ci, j)
            return carry

        lax.fori_loop(0, _NCHK // 2, pair_body, 0)
        wait_out(0)
        wait_out(1)

    return sc_kernel(x_shard, idx_prep)


def _tc_fuse(x2, xq, a, b_lo, b_hi, gamma, beta):
    def body(x2_ref, xq_ref, a_ref, bl_ref, bh_ref, g_ref, bt_ref, o_ref):
        xp = x2_ref[...]
        xw = xq_ref[:, pl.ds(0, _N)]
        xlo = lax.bitcast_convert_type(xw << 16, jnp.float32)
        xhi = lax.bitcast_convert_type(xw & jnp.int32(-65536), jnp.float32)
        nn = (((1,), (0,)), ((), ()))
        y = lax.dot_general(a_ref[...], xp, nn,
                            precision=lax.Precision.HIGHEST,
                            preferred_element_type=jnp.float32)
        y = y + lax.dot_general(bl_ref[...], xlo, nn,
                                precision=lax.Precision.HIGHEST,
                                preferred_element_type=jnp.float32)
        y = y + lax.dot_general(bh_ref[...], xhi, nn,
                                precision=lax.Precision.HIGHEST,
                                preferred_element_type=jnp.float32)
        m = jnp.mean(y, axis=1, keepdims=True)
        yc = y - m
        var = jnp.mean(yc * yc, axis=1, keepdims=True)
        scale = g_ref[...] * lax.rsqrt(var + 1e-5)
        o_ref[...] = jnp.maximum(yc * scale + bt_ref[...], 0.0)

    return pl.pallas_call(
        body,
        out_shape=jax.ShapeDtypeStruct((_C, _N), jnp.float32),
    )(x2, xq, a, b_lo, b_hi, gamma, beta)


def kernel(x, edge_index, conv_w, conv_b, bn_gamma, bn_beta):
    del conv_b
    x2 = x[0, :, :, 0]
    x_pad = jnp.pad(x2.T, ((0, _NPAD - _N), (0, 0)))
    x_pack = lax.bitcast_convert_type(
        x_pad.astype(jnp.bfloat16).reshape(_NPAD, _C // 2, 2), jnp.int32)
    x_shard = (x_pack.reshape(_NPAD, _NT, _WPT).transpose(1, 2, 0)
               .reshape(_NT, _NPAD * _WPT))
    ei_p = jnp.pad(edge_index, ((0, 0), (0, _NPAD - _N), (0, 0)))
    idx_prep = (ei_p.transpose(0, 2, 1)
                .reshape(2, _K, _NSC, _NCHK, _CN)
                .transpose(2, 3, 0, 1, 4))
    outp = _sc_max_rel(x_shard, idx_prep)
    xq = outp.reshape(_C // 2, _NPAD)
    w = conv_w.reshape(4, 32, 32, 2)
    eye = jnp.eye(4, dtype=conv_w.dtype)
    a = jnp.einsum('goc,gh->gohc', w[..., 0], eye).reshape(_C, _C)
    b = jnp.einsum('goc,gh->gohc', w[..., 1], eye).reshape(_C, _C)
    b_pairs = b.reshape(_C, _C // 2, 2)
    b_lo = b_pairs[:, :, 0]
    b_hi = b_pairs[:, :, 1]
    y = _tc_fuse(x2, xq, a, b_lo, b_hi,
                 bn_gamma.reshape(_C, 1), bn_beta.reshape(_C, 1))
    return y[None, :, :, None]
